# baseline (speedup 1.0000x reference)
import functools

import jax
import jax.numpy as jnp
from jax import lax
from jax.experimental import pallas as pl
from jax.experimental.pallas import tpu as pltpu
from jax.experimental.pallas import tpu_sc as plsc

N_NODES = 10000
N_EDGES = 320000
NC = 2
NS = 16
NW = NC * NS
EW = N_EDGES // NW
CH = 96
NCHUNK = 105
EWP = NCHUNK * CH
NB = 5
NP = 10240
ROWS_T = NP // NS
ZR = 128


def _mesh():
    return plsc.VectorSubcoreMesh(core_axis_name="c", subcore_axis_name="s",
                                  num_cores=NC, num_subcores=NS)


def _zero_fill(buf, nwords):
    zv = jnp.zeros((16,), jnp.float32)

    def body(i, _):
        buf[pl.ds(i * 16, 16)] = zv
        return 0

    lax.fori_loop(0, nwords // 16, body, 0)


def _make_segsum(d_feats):
    @functools.partial(
        pl.kernel,
        mesh=_mesh(),
        compiler_params=pltpu.CompilerParams(use_tc_tiling_on_sc=False),
        out_type=jax.ShapeDtypeStruct((NC, NP, d_feats), jnp.float32),
        scratch_types=[
            pltpu.VMEM((NCHUNK, CH), jnp.int32),
            pltpu.VMEM((NCHUNK, CH), jnp.int32),
            pltpu.VMEM((NB, CH, d_feats), jnp.float32),
            pltpu.VMEM((ZR, d_feats), jnp.float32),
            pltpu.VMEM_SHARED((NP, d_feats), jnp.float32),
            pltpu.SemaphoreType.DMA((NB,)),
            pltpu.SemaphoreType.DMA((NB,)),
        ],
    )
    def segsum(hp, er, out, src_v, dst_v, rows_v, zero_v, acc, gsem, ssem):
        c = lax.axis_index("c")
        s = lax.axis_index("s")
        w = c * NS + s

        zv = jnp.zeros((16,), jnp.float32)

        def zbody(i, _):
            for j in range(d_feats // 16):
                zero_v[i, pl.ds(j * 16, 16)] = zv
            return 0

        lax.fori_loop(0, ZR, zbody, 0)
        for k in range(ROWS_T // ZR):
            pltpu.sync_copy(zero_v, acc.at[pl.ds(s * ROWS_T + k * ZR, ZR)])

        pltpu.sync_copy(er.at[0, w], src_v)
        pltpu.sync_copy(er.at[1, w], dst_v)
        plsc.subcore_barrier()

        def wait_gather(b):
            pltpu.make_async_copy(hp.at[pl.ds(0, CH)], rows_v.at[b],
                                  gsem.at[b]).wait()

        def wait_scatter(b):
            pltpu.make_async_copy(rows_v.at[b], acc.at[pl.ds(0, CH)],
                                  ssem.at[b]).wait()

        for b in range(NB):
            pltpu.async_copy(hp.at[src_v.at[b]], rows_v.at[b], gsem.at[b])

        def body(g, _):
            c0 = g * NB
            for b in range(NB):
                wait_gather(b)
                pltpu.async_copy(rows_v.at[b], acc.at[dst_v.at[c0 + b]],
                                 ssem.at[b], add=True)
            for b in range(NB):
                wait_scatter(b)
                pltpu.async_copy(hp.at[src_v.at[c0 + NB + b]], rows_v.at[b],
                                 gsem.at[b])
            return 0

        lax.fori_loop(0, NCHUNK // NB - 1, body, 0)

        c0 = NCHUNK - NB
        for b in range(NB):
            wait_gather(b)
            pltpu.async_copy(rows_v.at[b], acc.at[dst_v.at[c0 + b]],
                             ssem.at[b], add=True)
        for b in range(NB):
            wait_scatter(b)

        plsc.subcore_barrier()
        pltpu.sync_copy(acc.at[pl.ds(s * ROWS_T, ROWS_T)],
                        out.at[c, pl.ds(s * ROWS_T, ROWS_T)])

    return segsum


def _make_hist():
    @functools.partial(
        pl.kernel,
        mesh=_mesh(),
        out_type=jax.ShapeDtypeStruct((NC, NP), jnp.float32),
        scratch_types=[
            pltpu.VMEM((NCHUNK, CH), jnp.int32),
            pltpu.VMEM((CH,), jnp.float32),
            pltpu.VMEM((ROWS_T,), jnp.float32),
            pltpu.VMEM_SHARED((NP,), jnp.float32),
        ],
    )
    def hist(er, out, dst_v, ones_v, zero_v, acc):
        c = lax.axis_index("c")
        s = lax.axis_index("s")
        w = c * NS + s

        _zero_fill(zero_v, ROWS_T)
        pltpu.sync_copy(zero_v, acc.at[pl.ds(s * ROWS_T, ROWS_T)])

        ov = jnp.ones((16,), jnp.float32)
        for i in range(CH // 16):
            ones_v[pl.ds(i * 16, 16)] = ov

        pltpu.sync_copy(er.at[1, w], dst_v)
        plsc.subcore_barrier()

        def body(i, _):
            pltpu.sync_copy(ones_v, acc.at[dst_v.at[i]], add=True)
            return 0

        lax.fori_loop(0, NCHUNK, body, 0)

        plsc.subcore_barrier()
        pltpu.sync_copy(acc.at[pl.ds(s * ROWS_T, ROWS_T)],
                        out.at[c, pl.ds(s * ROWS_T, ROWS_T)])

    return hist


def _stage_a1_body(x_ref, w1_ref, h_ref):
    h_ref[...] = jnp.dot(x_ref[...], w1_ref[...],
                         preferred_element_type=jnp.float32)


def _stage_a2_body(deg_ref, h_ref, hp_ref, dinv_ref):
    degT = jnp.transpose(deg_ref[...])[:N_NODES]
    deg = degT[:, 0:1] + degT[:, 1:2] + 1.0
    dinv = lax.rsqrt(deg)
    hp_ref[...] = h_ref[...] * dinv
    dinv_ref[...] = dinv


def _stage_b_body(raw_ref, hp_ref, dinv_ref, b1_ref, w2_ref, hp2_ref):
    dinv = dinv_ref[...]
    raw = raw_ref[0, :N_NODES] + raw_ref[1, :N_NODES]
    z = dinv * (raw + hp_ref[...]) + b1_ref[...]
    z = jnp.maximum(z, 0.0)
    hp2_ref[...] = jnp.dot(z, w2_ref[...],
                           preferred_element_type=jnp.float32) * dinv


def _stage_c_body(raw_ref, hp2_ref, dinv_ref, b2_ref, wn_ref, bn_ref, out_ref):
    dinv = dinv_ref[...]
    raw = raw_ref[0, :N_NODES] + raw_ref[1, :N_NODES]
    z = dinv * (raw + hp2_ref[...]) + b2_ref[...]
    z = jnp.maximum(z, 0.0)
    s = jnp.dot(z, wn_ref[...], preferred_element_type=jnp.float32)
    out_ref[...] = (s + bn_ref[...])[:, 0]


def kernel(x, edge_index, W1, b1, W2, b2, Wn, bn):
    ei2 = edge_index.reshape(2, NW, EW)
    npad = EWP - EW
    pad_dst = jnp.broadcast_to(
        (N_NODES + jnp.arange(npad, dtype=edge_index.dtype) % (NP - N_NODES)
         )[None, :], (NW, npad))
    fill = jnp.stack([jnp.zeros((NW, npad), edge_index.dtype), pad_dst])
    er = jnp.concatenate([ei2, fill], axis=2).reshape(2, NW, NCHUNK, CH)

    deg_parts = _make_hist()(er)
    h1, = pl.pallas_call(
        _stage_a1_body,
        out_shape=[jax.ShapeDtypeStruct((N_NODES, 64), jnp.float32)],
    )(x, W1)

    hp1, dinv = pl.pallas_call(
        _stage_a2_body,
        out_shape=[jax.ShapeDtypeStruct((N_NODES, 64), jnp.float32),
                   jax.ShapeDtypeStruct((N_NODES, 1), jnp.float32)],
    )(deg_parts, h1)

    raw1 = _make_segsum(64)(hp1, er)

    hp2, = pl.pallas_call(
        _stage_b_body,
        out_shape=[jax.ShapeDtypeStruct((N_NODES, 32), jnp.float32)],
    )(raw1, hp1, dinv, b1, W2)

    raw2 = _make_segsum(32)(hp2, er)

    scores, = pl.pallas_call(
        _stage_c_body,
        out_shape=[jax.ShapeDtypeStruct((N_NODES,), jnp.float32)],
    )(raw2, hp2, dinv, b2, Wn, bn)

    return scores


# speedup vs baseline: 1.3659x; 1.3659x over previous
import functools

import jax
import jax.numpy as jnp
from jax import lax
from jax.experimental import pallas as pl
from jax.experimental.pallas import tpu as pltpu
from jax.experimental.pallas import tpu_sc as plsc

N_NODES = 10000
N_EDGES = 320000
NC = 2
NS = 16
NW = NC * NS
EW = N_EDGES // NW
CH = 80
NCHUNK = 125
EWP = NCHUNK * CH
NB = 5
NP = 10240
ROWS_T = NP // NS
ZR = 128


def _mesh():
    return plsc.VectorSubcoreMesh(core_axis_name="c", subcore_axis_name="s",
                                  num_cores=NC, num_subcores=NS)


def _zero_fill(buf, nwords):
    zv = jnp.zeros((16,), jnp.float32)

    def body(i, _):
        buf[pl.ds(i * 16, 16)] = zv
        return 0

    lax.fori_loop(0, nwords // 16, body, 0)


def _make_segsum(d_feats):
    @functools.partial(
        pl.kernel,
        mesh=_mesh(),
        compiler_params=pltpu.CompilerParams(use_tc_tiling_on_sc=False),
        out_type=jax.ShapeDtypeStruct((NC, NP, d_feats), jnp.float32),
        scratch_types=[
            pltpu.VMEM((NCHUNK, CH), jnp.int32),
            pltpu.VMEM((NCHUNK, CH), jnp.int32),
            pltpu.VMEM((NB, CH, d_feats), jnp.float32),
            pltpu.VMEM((ZR, d_feats), jnp.float32),
            pltpu.VMEM_SHARED((NP, d_feats), jnp.float32),
            pltpu.SemaphoreType.DMA((NB,)),
            pltpu.SemaphoreType.DMA((NB,)),
        ],
    )
    def segsum(hp, er, out, src_v, dst_v, rows_v, zero_v, acc, gsem, ssem):
        c = lax.axis_index("c")
        s = lax.axis_index("s")
        w = c * NS + s

        zv = jnp.zeros((16,), jnp.float32)

        def zbody(i, _):
            for j in range(d_feats // 16):
                zero_v[i, pl.ds(j * 16, 16)] = zv
            return 0

        lax.fori_loop(0, ZR, zbody, 0)
        for k in range(ROWS_T // ZR):
            pltpu.sync_copy(zero_v, acc.at[pl.ds(s * ROWS_T + k * ZR, ZR)])

        pltpu.sync_copy(er.at[0, w], src_v)
        pltpu.sync_copy(er.at[1, w], dst_v)
        plsc.subcore_barrier()

        def wait_gather(b):
            pltpu.make_async_copy(hp.at[pl.ds(0, CH)], rows_v.at[b],
                                  gsem.at[b]).wait()

        def wait_scatter(b):
            pltpu.make_async_copy(rows_v.at[b], acc.at[pl.ds(0, CH)],
                                  ssem.at[b]).wait()

        for b in range(NB):
            pltpu.async_copy(hp.at[src_v.at[b]], rows_v.at[b], gsem.at[b])

        def body(g, _):
            c0 = g * NB
            for b in range(NB):
                wait_gather(b)
                pltpu.async_copy(rows_v.at[b], acc.at[dst_v.at[c0 + b]],
                                 ssem.at[b], add=True)
            for b in range(NB):
                wait_scatter(b)
                pltpu.async_copy(hp.at[src_v.at[c0 + NB + b]], rows_v.at[b],
                                 gsem.at[b])
            return 0

        lax.fori_loop(0, NCHUNK // NB - 1, body, 0)

        c0 = NCHUNK - NB
        for b in range(NB):
            wait_gather(b)
            pltpu.async_copy(rows_v.at[b], acc.at[dst_v.at[c0 + b]],
                             ssem.at[b], add=True)
        for b in range(NB):
            wait_scatter(b)

        plsc.subcore_barrier()
        pltpu.sync_copy(acc.at[pl.ds(s * ROWS_T, ROWS_T)],
                        out.at[c, pl.ds(s * ROWS_T, ROWS_T)])

    return segsum


def _make_hist():
    @functools.partial(
        pl.kernel,
        mesh=_mesh(),
        compiler_params=pltpu.CompilerParams(needs_layout_passes=False),
        out_type=jax.ShapeDtypeStruct((NC, NP), jnp.float32),
        scratch_types=[
            pltpu.VMEM((NCHUNK, CH), jnp.int32),
            pltpu.VMEM((NP,), jnp.float32),
            pltpu.VMEM((NS, ROWS_T), jnp.float32),
            pltpu.VMEM((ROWS_T,), jnp.float32),
            pltpu.VMEM_SHARED((NS, NP), jnp.float32),
        ],
    )
    def hist(er, out, dst_v, tab_v, slab_v, red_v, stage):
        c = lax.axis_index("c")
        s = lax.axis_index("s")
        w = c * NS + s

        _zero_fill(tab_v, NP)
        pltpu.sync_copy(er.at[1, w], dst_v)

        ones = jnp.ones((16,), jnp.float32)

        def body(i, _):
            for j in range(CH // 16):
                idx = dst_v[i, pl.ds(j * 16, 16)]
                plsc.addupdate_scatter(tab_v, [idx], ones)
            return 0

        lax.fori_loop(0, NCHUNK, body, 0)

        pltpu.sync_copy(tab_v, stage.at[s])
        plsc.subcore_barrier()

        pltpu.sync_copy(stage.at[:, pl.ds(s * ROWS_T, ROWS_T)], slab_v)

        def rbody(k, _):
            a = slab_v[0, pl.ds(k * 16, 16)]
            for t in range(1, NS):
                a = a + slab_v[t, pl.ds(k * 16, 16)]
            red_v[pl.ds(k * 16, 16)] = a
            return 0

        lax.fori_loop(0, ROWS_T // 16, rbody, 0)
        pltpu.sync_copy(red_v, out.at[c, pl.ds(s * ROWS_T, ROWS_T)])

    return hist


def _stage_a1_body(x_ref, w1_ref, h_ref):
    h_ref[...] = jnp.dot(x_ref[...], w1_ref[...],
                         preferred_element_type=jnp.float32)


def _stage_a2_body(deg_ref, h_ref, hp_ref, dinv_ref):
    degT = jnp.transpose(deg_ref[...])[:N_NODES]
    deg = degT[:, 0:1] + degT[:, 1:2] + 1.0
    dinv = lax.rsqrt(deg)
    hp_ref[...] = h_ref[...] * dinv
    dinv_ref[...] = dinv


def _stage_b_body(raw_ref, hp_ref, dinv_ref, b1_ref, w2_ref, hp2_ref):
    dinv = dinv_ref[...]
    raw = raw_ref[0, :N_NODES] + raw_ref[1, :N_NODES]
    z = dinv * (raw + hp_ref[...]) + b1_ref[...]
    z = jnp.maximum(z, 0.0)
    hp2_ref[...] = jnp.dot(z, w2_ref[...],
                           preferred_element_type=jnp.float32) * dinv


def _stage_c_body(raw_ref, hp2_ref, dinv_ref, b2_ref, wn_ref, bn_ref, out_ref):
    dinv = dinv_ref[...]
    raw = raw_ref[0, :N_NODES] + raw_ref[1, :N_NODES]
    z = dinv * (raw + hp2_ref[...]) + b2_ref[...]
    z = jnp.maximum(z, 0.0)
    s = jnp.dot(z, wn_ref[...], preferred_element_type=jnp.float32)
    out_ref[...] = (s + bn_ref[...])[:, 0]


def kernel(x, edge_index, W1, b1, W2, b2, Wn, bn):
    er = edge_index.reshape(2, NW, NCHUNK, CH)

    deg_parts = _make_hist()(er)
    h1, = pl.pallas_call(
        _stage_a1_body,
        out_shape=[jax.ShapeDtypeStruct((N_NODES, 64), jnp.float32)],
    )(x, W1)

    hp1, dinv = pl.pallas_call(
        _stage_a2_body,
        out_shape=[jax.ShapeDtypeStruct((N_NODES, 64), jnp.float32),
                   jax.ShapeDtypeStruct((N_NODES, 1), jnp.float32)],
    )(deg_parts, h1)

    raw1 = _make_segsum(64)(hp1, er)

    hp2, = pl.pallas_call(
        _stage_b_body,
        out_shape=[jax.ShapeDtypeStruct((N_NODES, 32), jnp.float32)],
    )(raw1, hp1, dinv, b1, W2)

    raw2 = _make_segsum(32)(hp2, er)

    scores, = pl.pallas_call(
        _stage_c_body,
        out_shape=[jax.ShapeDtypeStruct((N_NODES,), jnp.float32)],
    )(raw2, hp2, dinv, b2, Wn, bn)

    return scores
